# baseline (device time: 236960 ns/iter reference)
import jax
import jax.numpy as jnp
from jax import lax
from jax.experimental import pallas as pl
from jax.experimental.pallas import tpu as pltpu

N_DEV = 16


def kernel(t, W):
    m, k = t.shape
    _, n = W.shape
    chunk = m // N_DEV

    def body(t_ref, w_ref, out_ref, comm_ref, rs_send, rs_recv, ag_send, ag_recv):
        me = lax.axis_index("i")
        left = lax.rem(me + N_DEV - 1, N_DEV)
        right = lax.rem(me + 1, N_DEV)

        barrier = pltpu.get_barrier_semaphore()
        for nbr in (left, right):
            pl.semaphore_signal(
                barrier, inc=1, device_id=(nbr,),
                device_id_type=pl.DeviceIdType.MESH,
            )
        pl.semaphore_wait(barrier, 2)

        c0 = lax.rem(me + N_DEV - 1, N_DEV)
        comm_ref[N_DEV - 1, :, :] = t_ref[pl.ds(c0 * chunk, chunk), :]

        for h in range(N_DEV - 1):
            src_slot = N_DEV - 1 if h == 0 else h - 1
            rdma = pltpu.make_async_remote_copy(
                src_ref=comm_ref.at[src_slot],
                dst_ref=comm_ref.at[h],
                send_sem=rs_send.at[h],
                recv_sem=rs_recv.at[h],
                device_id=(right,),
                device_id_type=pl.DeviceIdType.MESH,
            )
            rdma.start()
            rdma.wait()
            r = lax.rem(me + 2 * N_DEV - h - 2, N_DEV)
            comm_ref[h, :, :] = comm_ref[h, :, :] + t_ref[pl.ds(r * chunk, chunk), :]

        out_ref[pl.ds(me * chunk, chunk), :] = jnp.dot(
            comm_ref[N_DEV - 2, :, :], w_ref[:, :],
            preferred_element_type=jnp.float32,
        )

        for g in range(N_DEV - 1):
            c = lax.rem(me + N_DEV - g, N_DEV)
            rdma = pltpu.make_async_remote_copy(
                src_ref=out_ref.at[pl.ds(c * chunk, chunk), :],
                dst_ref=out_ref.at[pl.ds(c * chunk, chunk), :],
                send_sem=ag_send.at[g],
                recv_sem=ag_recv.at[g],
                device_id=(right,),
                device_id_type=pl.DeviceIdType.MESH,
            )
            rdma.start()
            rdma.wait()

    return pl.pallas_call(
        body,
        out_shape=jax.ShapeDtypeStruct((m, n), jnp.float32),
        in_specs=[
            pl.BlockSpec(memory_space=pltpu.VMEM),
            pl.BlockSpec(memory_space=pltpu.VMEM),
        ],
        out_specs=pl.BlockSpec(memory_space=pltpu.VMEM),
        scratch_shapes=[
            pltpu.VMEM((N_DEV, chunk, k), jnp.float32),
            pltpu.SemaphoreType.DMA((N_DEV - 1,)),
            pltpu.SemaphoreType.DMA((N_DEV - 1,)),
            pltpu.SemaphoreType.DMA((N_DEV - 1,)),
            pltpu.SemaphoreType.DMA((N_DEV - 1,)),
        ],
        compiler_params=pltpu.CompilerParams(collective_id=0),
    )(t, W)


# device time: 193140 ns/iter; 1.2269x vs baseline; 1.2269x over previous
import jax
import jax.numpy as jnp
from jax import lax
from jax.experimental import pallas as pl
from jax.experimental.pallas import tpu as pltpu

N_DEV = 16


def kernel(t, W):
    m, k = t.shape
    _, n = W.shape
    chunk = m // N_DEV
    half = chunk // 2

    def body(t_ref, w_ref, out_ref, comm_r, comm_l,
             rs_send_r, rs_recv_r, rs_send_l, rs_recv_l,
             ag_send_r, ag_recv_r, ag_send_l, ag_recv_l):
        me = lax.axis_index("i")
        left = lax.rem(me + N_DEV - 1, N_DEV)
        right = lax.rem(me + 1, N_DEV)

        barrier = pltpu.get_barrier_semaphore()
        for nbr in (left, right):
            pl.semaphore_signal(
                barrier, inc=1, device_id=(nbr,),
                device_id_type=pl.DeviceIdType.MESH,
            )
        pl.semaphore_wait(barrier, 2)

        c0r = lax.rem(me + N_DEV - 1, N_DEV)
        c0l = lax.rem(me + 1, N_DEV)
        comm_r[N_DEV - 1, :, :] = t_ref[pl.ds(c0r * chunk, half), :]
        comm_l[N_DEV - 1, :, :] = t_ref[pl.ds(c0l * chunk + half, half), :]

        for h in range(N_DEV - 1):
            src_slot = N_DEV - 1 if h == 0 else h - 1
            rdma_r = pltpu.make_async_remote_copy(
                src_ref=comm_r.at[src_slot],
                dst_ref=comm_r.at[h],
                send_sem=rs_send_r.at[h],
                recv_sem=rs_recv_r.at[h],
                device_id=(right,),
                device_id_type=pl.DeviceIdType.MESH,
            )
            rdma_l = pltpu.make_async_remote_copy(
                src_ref=comm_l.at[src_slot],
                dst_ref=comm_l.at[h],
                send_sem=rs_send_l.at[h],
                recv_sem=rs_recv_l.at[h],
                device_id=(left,),
                device_id_type=pl.DeviceIdType.MESH,
            )
            rdma_r.start()
            rdma_l.start()
            rdma_r.wait()
            rdma_l.wait()
            rr = lax.rem(me + 2 * N_DEV - h - 2, N_DEV)
            rl = lax.rem(me + h + 2, N_DEV)
            comm_r[h, :, :] = comm_r[h, :, :] + t_ref[pl.ds(rr * chunk, half), :]
            comm_l[h, :, :] = comm_l[h, :, :] + t_ref[pl.ds(rl * chunk + half, half), :]

        out_ref[pl.ds(me * chunk, half), :] = jnp.dot(
            comm_r[N_DEV - 2, :, :], w_ref[:, :],
            preferred_element_type=jnp.float32,
        )
        out_ref[pl.ds(me * chunk + half, half), :] = jnp.dot(
            comm_l[N_DEV - 2, :, :], w_ref[:, :],
            preferred_element_type=jnp.float32,
        )

        for g in range(N_DEV - 1):
            cr = lax.rem(me + N_DEV - g, N_DEV)
            cl = lax.rem(me + g, N_DEV)
            rdma_r = pltpu.make_async_remote_copy(
                src_ref=out_ref.at[pl.ds(cr * chunk, half), :],
                dst_ref=out_ref.at[pl.ds(cr * chunk, half), :],
                send_sem=ag_send_r.at[g],
                recv_sem=ag_recv_r.at[g],
                device_id=(right,),
                device_id_type=pl.DeviceIdType.MESH,
            )
            rdma_l = pltpu.make_async_remote_copy(
                src_ref=out_ref.at[pl.ds(cl * chunk + half, half), :],
                dst_ref=out_ref.at[pl.ds(cl * chunk + half, half), :],
                send_sem=ag_send_l.at[g],
                recv_sem=ag_recv_l.at[g],
                device_id=(left,),
                device_id_type=pl.DeviceIdType.MESH,
            )
            rdma_r.start()
            rdma_l.start()
            rdma_r.wait()
            rdma_l.wait()

    return pl.pallas_call(
        body,
        out_shape=jax.ShapeDtypeStruct((m, n), jnp.float32),
        in_specs=[
            pl.BlockSpec(memory_space=pltpu.VMEM),
            pl.BlockSpec(memory_space=pltpu.VMEM),
        ],
        out_specs=pl.BlockSpec(memory_space=pltpu.VMEM),
        scratch_shapes=[
            pltpu.VMEM((N_DEV, half, k), jnp.float32),
            pltpu.VMEM((N_DEV, half, k), jnp.float32),
            pltpu.SemaphoreType.DMA((N_DEV - 1,)),
            pltpu.SemaphoreType.DMA((N_DEV - 1,)),
            pltpu.SemaphoreType.DMA((N_DEV - 1,)),
            pltpu.SemaphoreType.DMA((N_DEV - 1,)),
            pltpu.SemaphoreType.DMA((N_DEV - 1,)),
            pltpu.SemaphoreType.DMA((N_DEV - 1,)),
            pltpu.SemaphoreType.DMA((N_DEV - 1,)),
            pltpu.SemaphoreType.DMA((N_DEV - 1,)),
        ],
        compiler_params=pltpu.CompilerParams(collective_id=0),
    )(t, W)


# device time: 128640 ns/iter; 1.8420x vs baseline; 1.5014x over previous
import jax
import jax.numpy as jnp
from jax import lax
from jax.experimental import pallas as pl
from jax.experimental.pallas import tpu as pltpu

N_DEV = 16
S = 2


def kernel(t, W):
    m, k = t.shape
    _, n = W.shape
    chunk = m // N_DEV
    half = chunk // 2
    sub = half // S

    def body(t_ref, w_ref, out_ref, comm_r, comm_l,
             rs_send_r, rs_recv_r, rs_send_l, rs_recv_l,
             ag_send_r, ag_recv_r, ag_send_l, ag_recv_l):
        me = lax.axis_index("i")
        left = lax.rem(me + N_DEV - 1, N_DEV)
        right = lax.rem(me + 1, N_DEV)

        barrier = pltpu.get_barrier_semaphore()
        for nbr in (left, right):
            pl.semaphore_signal(
                barrier, inc=1, device_id=(nbr,),
                device_id_type=pl.DeviceIdType.MESH,
            )
        pl.semaphore_wait(barrier, 2)

        def rs_rdma(comm, send_sems, recv_sems, dst_dev, h, s):
            src_slot = N_DEV - 1 if h == 0 else h - 1
            return pltpu.make_async_remote_copy(
                src_ref=comm.at[src_slot, pl.ds(s * sub, sub), :],
                dst_ref=comm.at[h, pl.ds(s * sub, sub), :],
                send_sem=send_sems.at[h, s],
                recv_sem=recv_sems.at[h, s],
                device_id=(dst_dev,),
                device_id_type=pl.DeviceIdType.MESH,
            )

        c0r = lax.rem(me + N_DEV - 1, N_DEV)
        c0l = lax.rem(me + 1, N_DEV)
        for s in range(S):
            rows = pl.ds(s * sub, sub)
            comm_r[N_DEV - 1, rows, :] = t_ref[pl.ds(c0r * chunk + s * sub, sub), :]
            rs_rdma(comm_r, rs_send_r, rs_recv_r, right, 0, s).start()
            comm_l[N_DEV - 1, rows, :] = t_ref[pl.ds(c0l * chunk + half + s * sub, sub), :]
            rs_rdma(comm_l, rs_send_l, rs_recv_l, left, 0, s).start()

        for h in range(N_DEV - 1):
            rr = lax.rem(me + 2 * N_DEV - h - 2, N_DEV)
            rl = lax.rem(me + h + 2, N_DEV)
            for s in range(S):
                rows = pl.ds(s * sub, sub)
                rs_rdma(comm_r, rs_send_r, rs_recv_r, right, h, s).wait_recv()
                comm_r[h, rows, :] = (
                    comm_r[h, rows, :]
                    + t_ref[pl.ds(rr * chunk + s * sub, sub), :]
                )
                if h < N_DEV - 2:
                    rs_rdma(comm_r, rs_send_r, rs_recv_r, right, h + 1, s).start()
                rs_rdma(comm_l, rs_send_l, rs_recv_l, left, h, s).wait_recv()
                comm_l[h, rows, :] = (
                    comm_l[h, rows, :]
                    + t_ref[pl.ds(rl * chunk + half + s * sub, sub), :]
                )
                if h < N_DEV - 2:
                    rs_rdma(comm_l, rs_send_l, rs_recv_l, left, h + 1, s).start()

        out_ref[pl.ds(me * chunk, half), :] = jnp.dot(
            comm_r[N_DEV - 2, :, :], w_ref[:, :],
            preferred_element_type=jnp.float32,
        )
        out_ref[pl.ds(me * chunk + half, half), :] = jnp.dot(
            comm_l[N_DEV - 2, :, :], w_ref[:, :],
            preferred_element_type=jnp.float32,
        )

        def ag_rdma(send_sems, recv_sems, dst_dev, c, off, g, s):
            rows = pl.ds(c * chunk + off + s * sub, sub)
            return pltpu.make_async_remote_copy(
                src_ref=out_ref.at[rows, :],
                dst_ref=out_ref.at[rows, :],
                send_sem=send_sems.at[g, s],
                recv_sem=recv_sems.at[g, s],
                device_id=(dst_dev,),
                device_id_type=pl.DeviceIdType.MESH,
            )

        for s in range(S):
            ag_rdma(ag_send_r, ag_recv_r, right, me, 0, 0, s).start()
            ag_rdma(ag_send_l, ag_recv_l, left, me, half, 0, s).start()

        for g in range(N_DEV - 1):
            cr = lax.rem(me + 2 * N_DEV - g - 1, N_DEV)
            cl = lax.rem(me + g + 1, N_DEV)
            for s in range(S):
                ag_rdma(ag_send_r, ag_recv_r, right, cr, 0, g, s).wait_recv()
                if g < N_DEV - 2:
                    ag_rdma(ag_send_r, ag_recv_r, right, cr, 0, g + 1, s).start()
                ag_rdma(ag_send_l, ag_recv_l, left, cl, half, g, s).wait_recv()
                if g < N_DEV - 2:
                    ag_rdma(ag_send_l, ag_recv_l, left, cl, half, g + 1, s).start()

        for h in range(N_DEV - 1):
            for s in range(S):
                rs_rdma(comm_r, rs_send_r, rs_recv_r, right, h, s).wait_send()
                rs_rdma(comm_l, rs_send_l, rs_recv_l, left, h, s).wait_send()
                ag_rdma(ag_send_r, ag_recv_r, right, me, 0, h, s).wait_send()
                ag_rdma(ag_send_l, ag_recv_l, left, me, half, h, s).wait_send()

    return pl.pallas_call(
        body,
        out_shape=jax.ShapeDtypeStruct((m, n), jnp.float32),
        in_specs=[
            pl.BlockSpec(memory_space=pltpu.VMEM),
            pl.BlockSpec(memory_space=pltpu.VMEM),
        ],
        out_specs=pl.BlockSpec(memory_space=pltpu.VMEM),
        scratch_shapes=[
            pltpu.VMEM((N_DEV, half, k), jnp.float32),
            pltpu.VMEM((N_DEV, half, k), jnp.float32),
            pltpu.SemaphoreType.DMA((N_DEV - 1, S)),
            pltpu.SemaphoreType.DMA((N_DEV - 1, S)),
            pltpu.SemaphoreType.DMA((N_DEV - 1, S)),
            pltpu.SemaphoreType.DMA((N_DEV - 1, S)),
            pltpu.SemaphoreType.DMA((N_DEV - 1, S)),
            pltpu.SemaphoreType.DMA((N_DEV - 1, S)),
            pltpu.SemaphoreType.DMA((N_DEV - 1, S)),
            pltpu.SemaphoreType.DMA((N_DEV - 1, S)),
        ],
        compiler_params=pltpu.CompilerParams(collective_id=0),
    )(t, W)


# device time: 125592 ns/iter; 1.8867x vs baseline; 1.0243x over previous
import jax
import jax.numpy as jnp
from jax import lax
from jax.experimental import pallas as pl
from jax.experimental.pallas import tpu as pltpu

N_DEV = 16
S = 4


def kernel(t, W):
    m, k = t.shape
    _, n = W.shape
    chunk = m // N_DEV
    half = chunk // 2
    sub = half // S

    def body(t_ref, w_ref, out_ref, comm_r, comm_l,
             rs_send_r, rs_recv_r, rs_send_l, rs_recv_l,
             ag_send_r, ag_recv_r, ag_send_l, ag_recv_l):
        me = lax.axis_index("i")
        left = lax.rem(me + N_DEV - 1, N_DEV)
        right = lax.rem(me + 1, N_DEV)

        barrier = pltpu.get_barrier_semaphore()
        for nbr in (left, right):
            pl.semaphore_signal(
                barrier, inc=1, device_id=(nbr,),
                device_id_type=pl.DeviceIdType.MESH,
            )
        pl.semaphore_wait(barrier, 2)

        def rs_rdma(comm, send_sems, recv_sems, dst_dev, h, s):
            src_slot = N_DEV - 1 if h == 0 else h - 1
            return pltpu.make_async_remote_copy(
                src_ref=comm.at[src_slot, pl.ds(s * sub, sub), :],
                dst_ref=comm.at[h, pl.ds(s * sub, sub), :],
                send_sem=send_sems.at[h, s],
                recv_sem=recv_sems.at[h, s],
                device_id=(dst_dev,),
                device_id_type=pl.DeviceIdType.MESH,
            )

        c0r = lax.rem(me + N_DEV - 1, N_DEV)
        c0l = lax.rem(me + 1, N_DEV)
        for s in range(S):
            rows = pl.ds(s * sub, sub)
            comm_r[N_DEV - 1, rows, :] = t_ref[pl.ds(c0r * chunk + s * sub, sub), :]
            rs_rdma(comm_r, rs_send_r, rs_recv_r, right, 0, s).start()
            comm_l[N_DEV - 1, rows, :] = t_ref[pl.ds(c0l * chunk + half + s * sub, sub), :]
            rs_rdma(comm_l, rs_send_l, rs_recv_l, left, 0, s).start()

        for h in range(N_DEV - 1):
            rr = lax.rem(me + 2 * N_DEV - h - 2, N_DEV)
            rl = lax.rem(me + h + 2, N_DEV)
            for s in range(S):
                rows = pl.ds(s * sub, sub)
                rs_rdma(comm_r, rs_send_r, rs_recv_r, right, h, s).wait_recv()
                comm_r[h, rows, :] = (
                    comm_r[h, rows, :]
                    + t_ref[pl.ds(rr * chunk + s * sub, sub), :]
                )
                if h < N_DEV - 2:
                    rs_rdma(comm_r, rs_send_r, rs_recv_r, right, h + 1, s).start()
                rs_rdma(comm_l, rs_send_l, rs_recv_l, left, h, s).wait_recv()
                comm_l[h, rows, :] = (
                    comm_l[h, rows, :]
                    + t_ref[pl.ds(rl * chunk + half + s * sub, sub), :]
                )
                if h < N_DEV - 2:
                    rs_rdma(comm_l, rs_send_l, rs_recv_l, left, h + 1, s).start()

        out_ref[pl.ds(me * chunk, half), :] = jnp.dot(
            comm_r[N_DEV - 2, :, :], w_ref[:, :],
            preferred_element_type=jnp.float32,
        )
        out_ref[pl.ds(me * chunk + half, half), :] = jnp.dot(
            comm_l[N_DEV - 2, :, :], w_ref[:, :],
            preferred_element_type=jnp.float32,
        )

        def ag_rdma(send_sems, recv_sems, dst_dev, c, off, g, s):
            rows = pl.ds(c * chunk + off + s * sub, sub)
            return pltpu.make_async_remote_copy(
                src_ref=out_ref.at[rows, :],
                dst_ref=out_ref.at[rows, :],
                send_sem=send_sems.at[g, s],
                recv_sem=recv_sems.at[g, s],
                device_id=(dst_dev,),
                device_id_type=pl.DeviceIdType.MESH,
            )

        for s in range(S):
            ag_rdma(ag_send_r, ag_recv_r, right, me, 0, 0, s).start()
            ag_rdma(ag_send_l, ag_recv_l, left, me, half, 0, s).start()

        for g in range(N_DEV - 1):
            cr = lax.rem(me + 2 * N_DEV - g - 1, N_DEV)
            cl = lax.rem(me + g + 1, N_DEV)
            for s in range(S):
                ag_rdma(ag_send_r, ag_recv_r, right, cr, 0, g, s).wait_recv()
                if g < N_DEV - 2:
                    ag_rdma(ag_send_r, ag_recv_r, right, cr, 0, g + 1, s).start()
                ag_rdma(ag_send_l, ag_recv_l, left, cl, half, g, s).wait_recv()
                if g < N_DEV - 2:
                    ag_rdma(ag_send_l, ag_recv_l, left, cl, half, g + 1, s).start()

        for h in range(N_DEV - 1):
            for s in range(S):
                rs_rdma(comm_r, rs_send_r, rs_recv_r, right, h, s).wait_send()
                rs_rdma(comm_l, rs_send_l, rs_recv_l, left, h, s).wait_send()
                ag_rdma(ag_send_r, ag_recv_r, right, me, 0, h, s).wait_send()
                ag_rdma(ag_send_l, ag_recv_l, left, me, half, h, s).wait_send()

    return pl.pallas_call(
        body,
        out_shape=jax.ShapeDtypeStruct((m, n), jnp.float32),
        in_specs=[
            pl.BlockSpec(memory_space=pltpu.VMEM),
            pl.BlockSpec(memory_space=pltpu.VMEM),
        ],
        out_specs=pl.BlockSpec(memory_space=pltpu.VMEM),
        scratch_shapes=[
            pltpu.VMEM((N_DEV, half, k), jnp.float32),
            pltpu.VMEM((N_DEV, half, k), jnp.float32),
            pltpu.SemaphoreType.DMA((N_DEV - 1, S)),
            pltpu.SemaphoreType.DMA((N_DEV - 1, S)),
            pltpu.SemaphoreType.DMA((N_DEV - 1, S)),
            pltpu.SemaphoreType.DMA((N_DEV - 1, S)),
            pltpu.SemaphoreType.DMA((N_DEV - 1, S)),
            pltpu.SemaphoreType.DMA((N_DEV - 1, S)),
            pltpu.SemaphoreType.DMA((N_DEV - 1, S)),
            pltpu.SemaphoreType.DMA((N_DEV - 1, S)),
        ],
        compiler_params=pltpu.CompilerParams(collective_id=0),
    )(t, W)


# device time: 120704 ns/iter; 1.9631x vs baseline; 1.0405x over previous
import jax
import jax.numpy as jnp
from jax import lax
from jax.experimental import pallas as pl
from jax.experimental.pallas import tpu as pltpu

N_DEV = 16
S = 4


def kernel(t, W):
    m, k = t.shape
    _, n = W.shape
    chunk = m // N_DEV
    half = chunk // 2
    sub = half // S

    def body(t_ref, w_ref, out_ref, comm_r, comm_l, w_bf,
             rs_send_r, rs_recv_r, rs_send_l, rs_recv_l,
             ag_send_r, ag_recv_r, ag_send_l, ag_recv_l):
        me = lax.axis_index("i")
        left = lax.rem(me + N_DEV - 1, N_DEV)
        right = lax.rem(me + 1, N_DEV)

        barrier = pltpu.get_barrier_semaphore()
        for nbr in (left, right):
            pl.semaphore_signal(
                barrier, inc=1, device_id=(nbr,),
                device_id_type=pl.DeviceIdType.MESH,
            )
        pl.semaphore_wait(barrier, 2)

        def rs_rdma(comm, send_sems, recv_sems, dst_dev, h, s):
            src_slot = N_DEV - 1 if h == 0 else h - 1
            return pltpu.make_async_remote_copy(
                src_ref=comm.at[src_slot, pl.ds(s * sub, sub), :],
                dst_ref=comm.at[h, pl.ds(s * sub, sub), :],
                send_sem=send_sems.at[h, s],
                recv_sem=recv_sems.at[h, s],
                device_id=(dst_dev,),
                device_id_type=pl.DeviceIdType.MESH,
            )

        c0r = lax.rem(me + N_DEV - 1, N_DEV)
        c0l = lax.rem(me + 1, N_DEV)
        for s in range(S):
            rows = pl.ds(s * sub, sub)
            comm_r[N_DEV - 1, rows, :] = t_ref[
                pl.ds(c0r * chunk + s * sub, sub), :
            ].astype(jnp.bfloat16)
            rs_rdma(comm_r, rs_send_r, rs_recv_r, right, 0, s).start()
            comm_l[N_DEV - 1, rows, :] = t_ref[
                pl.ds(c0l * chunk + half + s * sub, sub), :
            ].astype(jnp.bfloat16)
            rs_rdma(comm_l, rs_send_l, rs_recv_l, left, 0, s).start()

        w_bf[:, :] = w_ref[:, :].astype(jnp.bfloat16)

        for h in range(N_DEV - 1):
            rr = lax.rem(me + 2 * N_DEV - h - 2, N_DEV)
            rl = lax.rem(me + h + 2, N_DEV)
            for s in range(S):
                rows = pl.ds(s * sub, sub)
                rs_rdma(comm_r, rs_send_r, rs_recv_r, right, h, s).wait_recv()
                comm_r[h, rows, :] = (
                    comm_r[h, rows, :].astype(jnp.float32)
                    + t_ref[pl.ds(rr * chunk + s * sub, sub), :]
                ).astype(jnp.bfloat16)
                if h < N_DEV - 2:
                    rs_rdma(comm_r, rs_send_r, rs_recv_r, right, h + 1, s).start()
                rs_rdma(comm_l, rs_send_l, rs_recv_l, left, h, s).wait_recv()
                comm_l[h, rows, :] = (
                    comm_l[h, rows, :].astype(jnp.float32)
                    + t_ref[pl.ds(rl * chunk + half + s * sub, sub), :]
                ).astype(jnp.bfloat16)
                if h < N_DEV - 2:
                    rs_rdma(comm_l, rs_send_l, rs_recv_l, left, h + 1, s).start()

        def ag_rdma(send_sems, recv_sems, dst_dev, c, off, g, s):
            rows = pl.ds(c * chunk + off + s * sub, sub)
            return pltpu.make_async_remote_copy(
                src_ref=out_ref.at[rows, :],
                dst_ref=out_ref.at[rows, :],
                send_sem=send_sems.at[g, s],
                recv_sem=recv_sems.at[g, s],
                device_id=(dst_dev,),
                device_id_type=pl.DeviceIdType.MESH,
            )

        out_ref[pl.ds(me * chunk, half), :] = jnp.dot(
            comm_r[N_DEV - 2, :, :], w_bf[:, :],
            preferred_element_type=jnp.float32,
        )
        for s in range(S):
            ag_rdma(ag_send_r, ag_recv_r, right, me, 0, 0, s).start()

        out_ref[pl.ds(me * chunk + half, half), :] = jnp.dot(
            comm_l[N_DEV - 2, :, :], w_bf[:, :],
            preferred_element_type=jnp.float32,
        )
        for s in range(S):
            ag_rdma(ag_send_l, ag_recv_l, left, me, half, 0, s).start()

        for g in range(N_DEV - 1):
            cr = lax.rem(me + 2 * N_DEV - g - 1, N_DEV)
            cl = lax.rem(me + g + 1, N_DEV)
            for s in range(S):
                ag_rdma(ag_send_r, ag_recv_r, right, cr, 0, g, s).wait_recv()
                if g < N_DEV - 2:
                    ag_rdma(ag_send_r, ag_recv_r, right, cr, 0, g + 1, s).start()
                ag_rdma(ag_send_l, ag_recv_l, left, cl, half, g, s).wait_recv()
                if g < N_DEV - 2:
                    ag_rdma(ag_send_l, ag_recv_l, left, cl, half, g + 1, s).start()

        for h in range(N_DEV - 1):
            for s in range(S):
                rs_rdma(comm_r, rs_send_r, rs_recv_r, right, h, s).wait_send()
                rs_rdma(comm_l, rs_send_l, rs_recv_l, left, h, s).wait_send()
                ag_rdma(ag_send_r, ag_recv_r, right, me, 0, h, s).wait_send()
                ag_rdma(ag_send_l, ag_recv_l, left, me, half, h, s).wait_send()

    return pl.pallas_call(
        body,
        out_shape=jax.ShapeDtypeStruct((m, n), jnp.float32),
        in_specs=[
            pl.BlockSpec(memory_space=pltpu.VMEM),
            pl.BlockSpec(memory_space=pltpu.VMEM),
        ],
        out_specs=pl.BlockSpec(memory_space=pltpu.VMEM),
        scratch_shapes=[
            pltpu.VMEM((N_DEV, half, k), jnp.bfloat16),
            pltpu.VMEM((N_DEV, half, k), jnp.bfloat16),
            pltpu.VMEM((k, n), jnp.bfloat16),
            pltpu.SemaphoreType.DMA((N_DEV - 1, S)),
            pltpu.SemaphoreType.DMA((N_DEV - 1, S)),
            pltpu.SemaphoreType.DMA((N_DEV - 1, S)),
            pltpu.SemaphoreType.DMA((N_DEV - 1, S)),
            pltpu.SemaphoreType.DMA((N_DEV - 1, S)),
            pltpu.SemaphoreType.DMA((N_DEV - 1, S)),
            pltpu.SemaphoreType.DMA((N_DEV - 1, S)),
            pltpu.SemaphoreType.DMA((N_DEV - 1, S)),
        ],
        compiler_params=pltpu.CompilerParams(collective_id=0),
    )(t, W)


# device time: 75466 ns/iter; 3.1400x vs baseline; 1.5994x over previous
import jax
import jax.numpy as jnp
from jax import lax
from jax.experimental import pallas as pl
from jax.experimental.pallas import tpu as pltpu

N_DEV = 16
G = 4
SA = 4
SB = 2
SE = 4
SUB = 64


def kernel(t, W):
    m, k = t.shape
    _, n = W.shape
    quarter = m // G
    qhalf = quarter // 2
    chunk = quarter // G

    def body(t_ref, w_ref, out_ref, a_r, a_l, q_buf, b_comm, qo_buf, po_buf,
             w_bf, a_send_r, a_recv_r, a_send_l, a_recv_l,
             b_send, b_recv, d_send, d_recv,
             e_send_r, e_recv_r, e_send_l, e_recv_l):
        me = lax.axis_index("i")
        q = lax.rem(me, G)
        zi = me // G
        base = me - q
        p_right = base + lax.rem(q + 1, G)
        p_left = base + lax.rem(q + 3, G)
        z_up = lax.rem(me + G, N_DEV)
        z_down = lax.rem(me + N_DEV - G, N_DEV)

        barrier = pltpu.get_barrier_semaphore()
        for nbr in (p_left, p_right, z_up, z_down):
            pl.semaphore_signal(
                barrier, inc=1, device_id=(nbr,),
                device_id_type=pl.DeviceIdType.MESH,
            )
        pl.semaphore_wait(barrier, 4)

        def a_rdma(comm, send_sems, recv_sems, dst, h, s):
            src_slot = G - 1 if h == 0 else h - 1
            return pltpu.make_async_remote_copy(
                src_ref=comm.at[src_slot, pl.ds(s * SUB, SUB), :],
                dst_ref=comm.at[h, pl.ds(s * SUB, SUB), :],
                send_sem=send_sems.at[h, s],
                recv_sem=recv_sems.at[h, s],
                device_id=(dst,),
                device_id_type=pl.DeviceIdType.MESH,
            )

        q0r = lax.rem(q + G - 1, G)
        q0l = lax.rem(q + 1, G)
        for s in range(SA):
            rows = pl.ds(s * SUB, SUB)
            a_r[G - 1, rows, :] = t_ref[
                pl.ds(q0r * quarter + s * SUB, SUB), :
            ].astype(jnp.bfloat16)
            a_rdma(a_r, a_send_r, a_recv_r, p_right, 0, s).start()
            a_l[G - 1, rows, :] = t_ref[
                pl.ds(q0l * quarter + qhalf + s * SUB, SUB), :
            ].astype(jnp.bfloat16)
            a_rdma(a_l, a_send_l, a_recv_l, p_left, 0, s).start()

        w_bf[:, :] = w_ref[:, :].astype(jnp.bfloat16)

        for h in range(G - 1):
            qr = lax.rem(q + 2 * G - h - 2, G)
            ql = lax.rem(q + h + 2, G)
            for s in range(SA):
                rows = pl.ds(s * SUB, SUB)
                a_rdma(a_r, a_send_r, a_recv_r, p_right, h, s).wait_recv()
                a_r[h, rows, :] = (
                    a_r[h, rows, :].astype(jnp.float32)
                    + t_ref[pl.ds(qr * quarter + s * SUB, SUB), :]
                ).astype(jnp.bfloat16)
                if h < G - 2:
                    a_rdma(a_r, a_send_r, a_recv_r, p_right, h + 1, s).start()
                a_rdma(a_l, a_send_l, a_recv_l, p_left, h, s).wait_recv()
                a_l[h, rows, :] = (
                    a_l[h, rows, :].astype(jnp.float32)
                    + t_ref[pl.ds(ql * quarter + qhalf + s * SUB, SUB), :]
                ).astype(jnp.bfloat16)
                if h < G - 2:
                    a_rdma(a_l, a_send_l, a_recv_l, p_left, h + 1, s).start()

        q_buf[0:qhalf, :] = a_r[G - 2, :, :]
        q_buf[qhalf:quarter, :] = a_l[G - 2, :, :]

        def b_rdma(h, s):
            src_slot = G - 1 if h == 0 else h - 1
            return pltpu.make_async_remote_copy(
                src_ref=b_comm.at[src_slot, pl.ds(s * SUB, SUB), :],
                dst_ref=b_comm.at[h, pl.ds(s * SUB, SUB), :],
                send_sem=b_send.at[h, s],
                recv_sem=b_recv.at[h, s],
                device_id=(z_up,),
                device_id_type=pl.DeviceIdType.MESH,
            )

        j0 = lax.rem(zi + G - 1, G)
        for s in range(SB):
            b_comm[G - 1, pl.ds(s * SUB, SUB), :] = q_buf[
                pl.ds(j0 * chunk + s * SUB, SUB), :
            ]
            b_rdma(0, s).start()
        for h in range(G - 1):
            jr = lax.rem(zi + 2 * G - h - 2, G)
            for s in range(SB):
                rows = pl.ds(s * SUB, SUB)
                b_rdma(h, s).wait_recv()
                b_comm[h, rows, :] = (
                    b_comm[h, rows, :].astype(jnp.float32)
                    + q_buf[pl.ds(jr * chunk + s * SUB, SUB), :].astype(jnp.float32)
                ).astype(jnp.bfloat16)
                if h < G - 2:
                    b_rdma(h + 1, s).start()

        res = jnp.dot(
            b_comm[G - 2, :, :], w_bf[:, :],
            preferred_element_type=jnp.float32,
        )
        out_ref[pl.ds(q * quarter + zi * chunk, chunk), :] = res
        qo_buf[pl.ds(zi * chunk, chunk), :] = res.astype(jnp.bfloat16)

        def d_rdma(c, g, s):
            rows = pl.ds(c * chunk + s * SUB, SUB)
            return pltpu.make_async_remote_copy(
                src_ref=qo_buf.at[rows, :],
                dst_ref=qo_buf.at[rows, :],
                send_sem=d_send.at[g, s],
                recv_sem=d_recv.at[g, s],
                device_id=(z_up,),
                device_id_type=pl.DeviceIdType.MESH,
            )

        for s in range(SB):
            d_rdma(zi, 0, s).start()
        for g in range(G - 1):
            jc = lax.rem(zi + 2 * G - g - 1, G)
            for s in range(SB):
                d_rdma(jc, g, s).wait_recv()
                if g < G - 2:
                    d_rdma(jc, g + 1, s).start()
                rows_o = pl.ds(q * quarter + jc * chunk + s * SUB, SUB)
                rows_q = pl.ds(jc * chunk + s * SUB, SUB)
                out_ref[rows_o, :] = qo_buf[rows_q, :].astype(jnp.float32)

        po_buf[pl.ds(q * quarter, quarter), :] = qo_buf[:, :]

        def e_rdma(send_sems, recv_sems, dst, c, off, g, s):
            rows = pl.ds(c * quarter + off + s * SUB, SUB)
            return pltpu.make_async_remote_copy(
                src_ref=po_buf.at[rows, :],
                dst_ref=po_buf.at[rows, :],
                send_sem=send_sems.at[g, s],
                recv_sem=recv_sems.at[g, s],
                device_id=(dst,),
                device_id_type=pl.DeviceIdType.MESH,
            )

        for s in range(SE):
            e_rdma(e_send_r, e_recv_r, p_right, q, 0, 0, s).start()
            e_rdma(e_send_l, e_recv_l, p_left, q, qhalf, 0, s).start()
        for g in range(G - 1):
            cr = lax.rem(q + 2 * G - g - 1, G)
            cl = lax.rem(q + g + 1, G)
            for s in range(SE):
                e_rdma(e_send_r, e_recv_r, p_right, cr, 0, g, s).wait_recv()
                if g < G - 2:
                    e_rdma(e_send_r, e_recv_r, p_right, cr, 0, g + 1, s).start()
                rows_r = pl.ds(cr * quarter + s * SUB, SUB)
                out_ref[rows_r, :] = po_buf[rows_r, :].astype(jnp.float32)
                e_rdma(e_send_l, e_recv_l, p_left, cl, qhalf, g, s).wait_recv()
                if g < G - 2:
                    e_rdma(e_send_l, e_recv_l, p_left, cl, qhalf, g + 1, s).start()
                rows_l = pl.ds(cl * quarter + qhalf + s * SUB, SUB)
                out_ref[rows_l, :] = po_buf[rows_l, :].astype(jnp.float32)

        for h in range(G - 1):
            for s in range(SA):
                a_rdma(a_r, a_send_r, a_recv_r, p_right, h, s).wait_send()
                a_rdma(a_l, a_send_l, a_recv_l, p_left, h, s).wait_send()
            for s in range(SB):
                b_rdma(h, s).wait_send()
                d_rdma(zi, h, s).wait_send()
            for s in range(SE):
                e_rdma(e_send_r, e_recv_r, p_right, q, 0, h, s).wait_send()
                e_rdma(e_send_l, e_recv_l, p_left, q, qhalf, h, s).wait_send()

    return pl.pallas_call(
        body,
        out_shape=jax.ShapeDtypeStruct((m, n), jnp.float32),
        in_specs=[
            pl.BlockSpec(memory_space=pltpu.VMEM),
            pl.BlockSpec(memory_space=pltpu.VMEM),
        ],
        out_specs=pl.BlockSpec(memory_space=pltpu.VMEM),
        scratch_shapes=[
            pltpu.VMEM((G, qhalf, k), jnp.bfloat16),
            pltpu.VMEM((G, qhalf, k), jnp.bfloat16),
            pltpu.VMEM((quarter, k), jnp.bfloat16),
            pltpu.VMEM((G, chunk, k), jnp.bfloat16),
            pltpu.VMEM((quarter, n), jnp.bfloat16),
            pltpu.VMEM((m, n), jnp.bfloat16),
            pltpu.VMEM((k, n), jnp.bfloat16),
            pltpu.SemaphoreType.DMA((G - 1, SA)),
            pltpu.SemaphoreType.DMA((G - 1, SA)),
            pltpu.SemaphoreType.DMA((G - 1, SA)),
            pltpu.SemaphoreType.DMA((G - 1, SA)),
            pltpu.SemaphoreType.DMA((G - 1, SB)),
            pltpu.SemaphoreType.DMA((G - 1, SB)),
            pltpu.SemaphoreType.DMA((G - 1, SB)),
            pltpu.SemaphoreType.DMA((G - 1, SB)),
            pltpu.SemaphoreType.DMA((G - 1, SE)),
            pltpu.SemaphoreType.DMA((G - 1, SE)),
            pltpu.SemaphoreType.DMA((G - 1, SE)),
            pltpu.SemaphoreType.DMA((G - 1, SE)),
        ],
        compiler_params=pltpu.CompilerParams(collective_id=0),
    )(t, W)


# device time: 72638 ns/iter; 3.2622x vs baseline; 1.0389x over previous
import jax
import jax.numpy as jnp
from jax import lax
from jax.experimental import pallas as pl
from jax.experimental.pallas import tpu as pltpu

N_DEV = 16
G = 4
SA = 4
SB = 2
SE = 4
SUB = 64


def kernel(t, W):
    m, k = t.shape
    _, n = W.shape
    quarter = m // G
    qhalf = quarter // 2
    chunk = quarter // G

    def body(t_ref, w_ref, out_ref, a_r, a_l, q_buf, b_comm, qo_buf, po_buf,
             w_bf, a_send_r, a_recv_r, a_send_l, a_recv_l,
             b_send, b_recv, d_send, d_recv,
             e_send_r, e_recv_r, e_send_l, e_recv_l):
        me = lax.axis_index("i")
        q = lax.rem(me, G)
        zi = me // G
        base = me - q
        p_right = base + lax.rem(q + 1, G)
        p_left = base + lax.rem(q + 3, G)
        z_up = lax.rem(me + G, N_DEV)
        z_down = lax.rem(me + N_DEV - G, N_DEV)

        barrier = pltpu.get_barrier_semaphore()
        for nbr in (p_left, p_right, z_up, z_down):
            pl.semaphore_signal(
                barrier, inc=1, device_id=(nbr,),
                device_id_type=pl.DeviceIdType.MESH,
            )
        pl.semaphore_wait(barrier, 4)

        def a_rdma(comm, send_sems, recv_sems, dst, h, s):
            src_slot = G - 1 if h == 0 else h - 1
            return pltpu.make_async_remote_copy(
                src_ref=comm.at[src_slot, pl.ds(s * SUB, SUB), :],
                dst_ref=comm.at[h, pl.ds(s * SUB, SUB), :],
                send_sem=send_sems.at[h, s],
                recv_sem=recv_sems.at[h, s],
                device_id=(dst,),
                device_id_type=pl.DeviceIdType.MESH,
            )

        q0r = lax.rem(q + G - 1, G)
        q0l = lax.rem(q + 1, G)
        for s in range(SA):
            rows = pl.ds(s * SUB, SUB)
            a_r[G - 1, rows, :] = t_ref[
                pl.ds(q0r * quarter + s * SUB, SUB), :
            ].astype(jnp.bfloat16)
            a_rdma(a_r, a_send_r, a_recv_r, p_right, 0, s).start()
            a_l[G - 1, rows, :] = t_ref[
                pl.ds(q0l * quarter + qhalf + s * SUB, SUB), :
            ].astype(jnp.bfloat16)
            a_rdma(a_l, a_send_l, a_recv_l, p_left, 0, s).start()

        w_bf[:, :] = w_ref[:, :].astype(jnp.bfloat16)

        for h in range(G - 1):
            qr = lax.rem(q + 2 * G - h - 2, G)
            ql = lax.rem(q + h + 2, G)
            for s in range(SA):
                rows = pl.ds(s * SUB, SUB)
                a_rdma(a_r, a_send_r, a_recv_r, p_right, h, s).wait_recv()
                a_r[h, rows, :] = (
                    a_r[h, rows, :].astype(jnp.float32)
                    + t_ref[pl.ds(qr * quarter + s * SUB, SUB), :]
                ).astype(jnp.bfloat16)
                if h < G - 2:
                    a_rdma(a_r, a_send_r, a_recv_r, p_right, h + 1, s).start()
                a_rdma(a_l, a_send_l, a_recv_l, p_left, h, s).wait_recv()
                a_l[h, rows, :] = (
                    a_l[h, rows, :].astype(jnp.float32)
                    + t_ref[pl.ds(ql * quarter + qhalf + s * SUB, SUB), :]
                ).astype(jnp.bfloat16)
                if h < G - 2:
                    a_rdma(a_l, a_send_l, a_recv_l, p_left, h + 1, s).start()

        q_buf[0:qhalf, :] = a_r[G - 2, :, :]
        q_buf[qhalf:quarter, :] = a_l[G - 2, :, :]

        def b_rdma(h, s):
            src_slot = G - 1 if h == 0 else h - 1
            return pltpu.make_async_remote_copy(
                src_ref=b_comm.at[src_slot, pl.ds(s * SUB, SUB), :],
                dst_ref=b_comm.at[h, pl.ds(s * SUB, SUB), :],
                send_sem=b_send.at[h, s],
                recv_sem=b_recv.at[h, s],
                device_id=(z_up,),
                device_id_type=pl.DeviceIdType.MESH,
            )

        j0 = lax.rem(zi + G - 1, G)
        for s in range(SB):
            b_comm[G - 1, pl.ds(s * SUB, SUB), :] = q_buf[
                pl.ds(j0 * chunk + s * SUB, SUB), :
            ]
            b_rdma(0, s).start()
        for h in range(G - 1):
            jr = lax.rem(zi + 2 * G - h - 2, G)
            for s in range(SB):
                rows = pl.ds(s * SUB, SUB)
                b_rdma(h, s).wait_recv()
                b_comm[h, rows, :] = (
                    b_comm[h, rows, :].astype(jnp.float32)
                    + q_buf[pl.ds(jr * chunk + s * SUB, SUB), :].astype(jnp.float32)
                ).astype(jnp.bfloat16)
                if h < G - 2:
                    b_rdma(h + 1, s).start()

        res = jnp.dot(
            b_comm[G - 2, :, :], w_bf[:, :],
            preferred_element_type=jnp.float32,
        )
        out_ref[pl.ds(q * quarter + zi * chunk, chunk), :] = res
        qo_buf[pl.ds(zi * chunk, chunk), :] = res.astype(jnp.bfloat16)

        def d_rdma(c, g, s):
            rows = pl.ds(c * chunk + s * SUB, SUB)
            return pltpu.make_async_remote_copy(
                src_ref=qo_buf.at[rows, :],
                dst_ref=qo_buf.at[rows, :],
                send_sem=d_send.at[g, s],
                recv_sem=d_recv.at[g, s],
                device_id=(z_up,),
                device_id_type=pl.DeviceIdType.MESH,
            )

        def e_rdma(send_sems, recv_sems, dst, c, off, g, s):
            rows = pl.ds(c * quarter + off + s * SUB, SUB)
            return pltpu.make_async_remote_copy(
                src_ref=po_buf.at[rows, :],
                dst_ref=po_buf.at[rows, :],
                send_sem=send_sems.at[g, s],
                recv_sem=recv_sems.at[g, s],
                device_id=(dst,),
                device_id_type=pl.DeviceIdType.MESH,
            )

        def e0_rdma(send_sems, recv_sems, dst, off, s):
            return pltpu.make_async_remote_copy(
                src_ref=qo_buf.at[pl.ds(off + s * SUB, SUB), :],
                dst_ref=po_buf.at[pl.ds(q * quarter + off + s * SUB, SUB), :],
                send_sem=send_sems.at[0, s],
                recv_sem=recv_sems.at[0, s],
                device_id=(dst,),
                device_id_type=pl.DeviceIdType.MESH,
            )

        def start_e0_for_chunk(x):
            for s in range(SE):
                @pl.when(x == s // 2)
                def _():
                    e0_rdma(e_send_r, e_recv_r, p_right, 0, s).start()

                @pl.when(x == 2 + s // 2)
                def _():
                    e0_rdma(e_send_l, e_recv_l, p_left, qhalf, s).start()

        for s in range(SB):
            d_rdma(zi, 0, s).start()
        start_e0_for_chunk(zi)
        for g in range(G - 1):
            jc = lax.rem(zi + 2 * G - g - 1, G)
            for s in range(SB):
                d_rdma(jc, g, s).wait_recv()
                if g < G - 2:
                    d_rdma(jc, g + 1, s).start()
            start_e0_for_chunk(jc)
            for s in range(SB):
                rows_o = pl.ds(q * quarter + jc * chunk + s * SUB, SUB)
                rows_q = pl.ds(jc * chunk + s * SUB, SUB)
                out_ref[rows_o, :] = qo_buf[rows_q, :].astype(jnp.float32)

        for g in range(G - 1):
            cr = lax.rem(q + 2 * G - g - 1, G)
            cl = lax.rem(q + g + 1, G)
            for s in range(SE):
                e_rdma(e_send_r, e_recv_r, p_right, cr, 0, g, s).wait_recv()
                if g < G - 2:
                    e_rdma(e_send_r, e_recv_r, p_right, cr, 0, g + 1, s).start()
                rows_r = pl.ds(cr * quarter + s * SUB, SUB)
                out_ref[rows_r, :] = po_buf[rows_r, :].astype(jnp.float32)
                e_rdma(e_send_l, e_recv_l, p_left, cl, qhalf, g, s).wait_recv()
                if g < G - 2:
                    e_rdma(e_send_l, e_recv_l, p_left, cl, qhalf, g + 1, s).start()
                rows_l = pl.ds(cl * quarter + qhalf + s * SUB, SUB)
                out_ref[rows_l, :] = po_buf[rows_l, :].astype(jnp.float32)

        for h in range(G - 1):
            for s in range(SA):
                a_rdma(a_r, a_send_r, a_recv_r, p_right, h, s).wait_send()
                a_rdma(a_l, a_send_l, a_recv_l, p_left, h, s).wait_send()
            for s in range(SB):
                b_rdma(h, s).wait_send()
                d_rdma(zi, h, s).wait_send()
            for s in range(SE):
                e_rdma(e_send_r, e_recv_r, p_right, q, 0, h, s).wait_send()
                e_rdma(e_send_l, e_recv_l, p_left, q, qhalf, h, s).wait_send()

    return pl.pallas_call(
        body,
        out_shape=jax.ShapeDtypeStruct((m, n), jnp.float32),
        in_specs=[
            pl.BlockSpec(memory_space=pltpu.VMEM),
            pl.BlockSpec(memory_space=pltpu.VMEM),
        ],
        out_specs=pl.BlockSpec(memory_space=pltpu.VMEM),
        scratch_shapes=[
            pltpu.VMEM((G, qhalf, k), jnp.bfloat16),
            pltpu.VMEM((G, qhalf, k), jnp.bfloat16),
            pltpu.VMEM((quarter, k), jnp.bfloat16),
            pltpu.VMEM((G, chunk, k), jnp.bfloat16),
            pltpu.VMEM((quarter, n), jnp.bfloat16),
            pltpu.VMEM((m, n), jnp.bfloat16),
            pltpu.VMEM((k, n), jnp.bfloat16),
            pltpu.SemaphoreType.DMA((G - 1, SA)),
            pltpu.SemaphoreType.DMA((G - 1, SA)),
            pltpu.SemaphoreType.DMA((G - 1, SA)),
            pltpu.SemaphoreType.DMA((G - 1, SA)),
            pltpu.SemaphoreType.DMA((G - 1, SB)),
            pltpu.SemaphoreType.DMA((G - 1, SB)),
            pltpu.SemaphoreType.DMA((G - 1, SB)),
            pltpu.SemaphoreType.DMA((G - 1, SB)),
            pltpu.SemaphoreType.DMA((G - 1, SE)),
            pltpu.SemaphoreType.DMA((G - 1, SE)),
            pltpu.SemaphoreType.DMA((G - 1, SE)),
            pltpu.SemaphoreType.DMA((G - 1, SE)),
        ],
        compiler_params=pltpu.CompilerParams(collective_id=0),
    )(t, W)


# device time: 71761 ns/iter; 3.3021x vs baseline; 1.0122x over previous
import jax
import jax.numpy as jnp
from jax import lax
from jax.experimental import pallas as pl
from jax.experimental.pallas import tpu as pltpu

N_DEV = 16
G = 4
SA = 8
SB = 4
SE = 8
SUB = 32


def kernel(t, W):
    m, k = t.shape
    _, n = W.shape
    quarter = m // G
    qhalf = quarter // 2
    chunk = quarter // G

    def body(t_ref, w_ref, out_ref, a_r, a_l, q_buf, b_comm, qo_buf, po_buf,
             w_bf, a_send_r, a_recv_r, a_send_l, a_recv_l,
             b_send, b_recv, d_send, d_recv,
             e_send_r, e_recv_r, e_send_l, e_recv_l):
        me = lax.axis_index("i")
        q = lax.rem(me, G)
        zi = me // G
        base = me - q
        p_right = base + lax.rem(q + 1, G)
        p_left = base + lax.rem(q + 3, G)
        z_up = lax.rem(me + G, N_DEV)
        z_down = lax.rem(me + N_DEV - G, N_DEV)

        barrier = pltpu.get_barrier_semaphore()
        for nbr in (p_left, p_right, z_up, z_down):
            pl.semaphore_signal(
                barrier, inc=1, device_id=(nbr,),
                device_id_type=pl.DeviceIdType.MESH,
            )
        pl.semaphore_wait(barrier, 4)

        def a_rdma(comm, send_sems, recv_sems, dst, h, s):
            src_slot = G - 1 if h == 0 else h - 1
            return pltpu.make_async_remote_copy(
                src_ref=comm.at[src_slot, pl.ds(s * SUB, SUB), :],
                dst_ref=comm.at[h, pl.ds(s * SUB, SUB), :],
                send_sem=send_sems.at[h, s],
                recv_sem=recv_sems.at[h, s],
                device_id=(dst,),
                device_id_type=pl.DeviceIdType.MESH,
            )

        q0r = lax.rem(q + G - 1, G)
        q0l = lax.rem(q + 1, G)
        for s in range(SA):
            rows = pl.ds(s * SUB, SUB)
            a_r[G - 1, rows, :] = t_ref[
                pl.ds(q0r * quarter + s * SUB, SUB), :
            ].astype(jnp.bfloat16)
            a_rdma(a_r, a_send_r, a_recv_r, p_right, 0, s).start()
            a_l[G - 1, rows, :] = t_ref[
                pl.ds(q0l * quarter + qhalf + s * SUB, SUB), :
            ].astype(jnp.bfloat16)
            a_rdma(a_l, a_send_l, a_recv_l, p_left, 0, s).start()

        w_bf[:, :] = w_ref[:, :].astype(jnp.bfloat16)

        for h in range(G - 1):
            qr = lax.rem(q + 2 * G - h - 2, G)
            ql = lax.rem(q + h + 2, G)
            for s in range(SA):
                rows = pl.ds(s * SUB, SUB)
                a_rdma(a_r, a_send_r, a_recv_r, p_right, h, s).wait_recv()
                a_r[h, rows, :] = (
                    a_r[h, rows, :].astype(jnp.float32)
                    + t_ref[pl.ds(qr * quarter + s * SUB, SUB), :]
                ).astype(jnp.bfloat16)
                if h < G - 2:
                    a_rdma(a_r, a_send_r, a_recv_r, p_right, h + 1, s).start()
                a_rdma(a_l, a_send_l, a_recv_l, p_left, h, s).wait_recv()
                a_l[h, rows, :] = (
                    a_l[h, rows, :].astype(jnp.float32)
                    + t_ref[pl.ds(ql * quarter + qhalf + s * SUB, SUB), :]
                ).astype(jnp.bfloat16)
                if h < G - 2:
                    a_rdma(a_l, a_send_l, a_recv_l, p_left, h + 1, s).start()

        q_buf[0:qhalf, :] = a_r[G - 2, :, :]
        q_buf[qhalf:quarter, :] = a_l[G - 2, :, :]

        def b_rdma(h, s):
            src_slot = G - 1 if h == 0 else h - 1
            return pltpu.make_async_remote_copy(
                src_ref=b_comm.at[src_slot, pl.ds(s * SUB, SUB), :],
                dst_ref=b_comm.at[h, pl.ds(s * SUB, SUB), :],
                send_sem=b_send.at[h, s],
                recv_sem=b_recv.at[h, s],
                device_id=(z_up,),
                device_id_type=pl.DeviceIdType.MESH,
            )

        j0 = lax.rem(zi + G - 1, G)
        for s in range(SB):
            b_comm[G - 1, pl.ds(s * SUB, SUB), :] = q_buf[
                pl.ds(j0 * chunk + s * SUB, SUB), :
            ]
            b_rdma(0, s).start()
        for h in range(G - 1):
            jr = lax.rem(zi + 2 * G - h - 2, G)
            for s in range(SB):
                rows = pl.ds(s * SUB, SUB)
                b_rdma(h, s).wait_recv()
                b_comm[h, rows, :] = (
                    b_comm[h, rows, :].astype(jnp.float32)
                    + q_buf[pl.ds(jr * chunk + s * SUB, SUB), :].astype(jnp.float32)
                ).astype(jnp.bfloat16)
                if h < G - 2:
                    b_rdma(h + 1, s).start()

        res = jnp.dot(
            b_comm[G - 2, :, :], w_bf[:, :],
            preferred_element_type=jnp.float32,
        )
        out_ref[pl.ds(q * quarter + zi * chunk, chunk), :] = res
        qo_buf[pl.ds(zi * chunk, chunk), :] = res.astype(jnp.bfloat16)

        def d_rdma(c, g, s):
            rows = pl.ds(c * chunk + s * SUB, SUB)
            return pltpu.make_async_remote_copy(
                src_ref=qo_buf.at[rows, :],
                dst_ref=qo_buf.at[rows, :],
                send_sem=d_send.at[g, s],
                recv_sem=d_recv.at[g, s],
                device_id=(z_up,),
                device_id_type=pl.DeviceIdType.MESH,
            )

        def e_rdma(send_sems, recv_sems, dst, c, off, g, s):
            rows = pl.ds(c * quarter + off + s * SUB, SUB)
            return pltpu.make_async_remote_copy(
                src_ref=po_buf.at[rows, :],
                dst_ref=po_buf.at[rows, :],
                send_sem=send_sems.at[g, s],
                recv_sem=recv_sems.at[g, s],
                device_id=(dst,),
                device_id_type=pl.DeviceIdType.MESH,
            )

        def e0_rdma(send_sems, recv_sems, dst, off, s):
            return pltpu.make_async_remote_copy(
                src_ref=qo_buf.at[pl.ds(off + s * SUB, SUB), :],
                dst_ref=po_buf.at[pl.ds(q * quarter + off + s * SUB, SUB), :],
                send_sem=send_sems.at[0, s],
                recv_sem=recv_sems.at[0, s],
                device_id=(dst,),
                device_id_type=pl.DeviceIdType.MESH,
            )

        def start_e0_for_chunk(x):
            for s in range(SE):
                @pl.when(x == (s * SUB) // chunk)
                def _():
                    e0_rdma(e_send_r, e_recv_r, p_right, 0, s).start()

                @pl.when(x == 2 + (s * SUB) // chunk)
                def _():
                    e0_rdma(e_send_l, e_recv_l, p_left, qhalf, s).start()

        for s in range(SB):
            d_rdma(zi, 0, s).start()
        start_e0_for_chunk(zi)
        for g in range(G - 1):
            jc = lax.rem(zi + 2 * G - g - 1, G)
            for s in range(SB):
                d_rdma(jc, g, s).wait_recv()
                if g < G - 2:
                    d_rdma(jc, g + 1, s).start()
            start_e0_for_chunk(jc)
            for s in range(SB):
                rows_o = pl.ds(q * quarter + jc * chunk + s * SUB, SUB)
                rows_q = pl.ds(jc * chunk + s * SUB, SUB)
                out_ref[rows_o, :] = qo_buf[rows_q, :].astype(jnp.float32)

        for g in range(G - 1):
            cr = lax.rem(q + 2 * G - g - 1, G)
            cl = lax.rem(q + g + 1, G)
            for s in range(SE):
                e_rdma(e_send_r, e_recv_r, p_right, cr, 0, g, s).wait_recv()
                if g < G - 2:
                    e_rdma(e_send_r, e_recv_r, p_right, cr, 0, g + 1, s).start()
                rows_r = pl.ds(cr * quarter + s * SUB, SUB)
                out_ref[rows_r, :] = po_buf[rows_r, :].astype(jnp.float32)
                e_rdma(e_send_l, e_recv_l, p_left, cl, qhalf, g, s).wait_recv()
                if g < G - 2:
                    e_rdma(e_send_l, e_recv_l, p_left, cl, qhalf, g + 1, s).start()
                rows_l = pl.ds(cl * quarter + qhalf + s * SUB, SUB)
                out_ref[rows_l, :] = po_buf[rows_l, :].astype(jnp.float32)

        for h in range(G - 1):
            for s in range(SA):
                a_rdma(a_r, a_send_r, a_recv_r, p_right, h, s).wait_send()
                a_rdma(a_l, a_send_l, a_recv_l, p_left, h, s).wait_send()
            for s in range(SB):
                b_rdma(h, s).wait_send()
                d_rdma(zi, h, s).wait_send()
            for s in range(SE):
                e_rdma(e_send_r, e_recv_r, p_right, q, 0, h, s).wait_send()
                e_rdma(e_send_l, e_recv_l, p_left, q, qhalf, h, s).wait_send()

    return pl.pallas_call(
        body,
        out_shape=jax.ShapeDtypeStruct((m, n), jnp.float32),
        in_specs=[
            pl.BlockSpec(memory_space=pltpu.VMEM),
            pl.BlockSpec(memory_space=pltpu.VMEM),
        ],
        out_specs=pl.BlockSpec(memory_space=pltpu.VMEM),
        scratch_shapes=[
            pltpu.VMEM((G, qhalf, k), jnp.bfloat16),
            pltpu.VMEM((G, qhalf, k), jnp.bfloat16),
            pltpu.VMEM((quarter, k), jnp.bfloat16),
            pltpu.VMEM((G, chunk, k), jnp.bfloat16),
            pltpu.VMEM((quarter, n), jnp.bfloat16),
            pltpu.VMEM((m, n), jnp.bfloat16),
            pltpu.VMEM((k, n), jnp.bfloat16),
            pltpu.SemaphoreType.DMA((G - 1, SA)),
            pltpu.SemaphoreType.DMA((G - 1, SA)),
            pltpu.SemaphoreType.DMA((G - 1, SA)),
            pltpu.SemaphoreType.DMA((G - 1, SA)),
            pltpu.SemaphoreType.DMA((G - 1, SB)),
            pltpu.SemaphoreType.DMA((G - 1, SB)),
            pltpu.SemaphoreType.DMA((G - 1, SB)),
            pltpu.SemaphoreType.DMA((G - 1, SB)),
            pltpu.SemaphoreType.DMA((G - 1, SE)),
            pltpu.SemaphoreType.DMA((G - 1, SE)),
            pltpu.SemaphoreType.DMA((G - 1, SE)),
            pltpu.SemaphoreType.DMA((G - 1, SE)),
        ],
        compiler_params=pltpu.CompilerParams(collective_id=0),
    )(t, W)


# device time: 71703 ns/iter; 3.3047x vs baseline; 1.0008x over previous
import jax
import jax.numpy as jnp
from jax import lax
from jax.experimental import pallas as pl
from jax.experimental.pallas import tpu as pltpu

N_DEV = 16
G = 4
SA = 8
SB = 4
SE = 8
SUB = 32


def kernel(t, W):
    m, k = t.shape
    _, n = W.shape
    quarter = m // G
    qhalf = quarter // 2
    chunk = quarter // G

    def body(t_ref, w_ref, out_ref, a_r, a_l, q_buf, b_comm, qo_buf, po_buf,
             w_bf, a_send_r, a_recv_r, a_send_l, a_recv_l,
             b_send, b_recv, d_send, d_recv,
             e_send_r, e_recv_r, e_send_l, e_recv_l):
        me = lax.axis_index("i")
        q = lax.rem(me, G)
        zi = me // G
        base = me - q
        p_right = base + lax.rem(q + 1, G)
        p_left = base + lax.rem(q + 3, G)
        z_up = lax.rem(me + G, N_DEV)
        z_down = lax.rem(me + N_DEV - G, N_DEV)

        barrier = pltpu.get_barrier_semaphore()
        for nbr in (p_left, p_right, z_up, z_down):
            pl.semaphore_signal(
                barrier, inc=1, device_id=(nbr,),
                device_id_type=pl.DeviceIdType.MESH,
            )
        pl.semaphore_wait(barrier, 4)

        def a_rdma(comm, send_sems, recv_sems, dst, h, s):
            src_slot = G - 1 if h == 0 else h - 1
            return pltpu.make_async_remote_copy(
                src_ref=comm.at[src_slot, pl.ds(s * SUB, SUB), :],
                dst_ref=comm.at[h, pl.ds(s * SUB, SUB), :],
                send_sem=send_sems.at[h, s],
                recv_sem=recv_sems.at[h, s],
                device_id=(dst,),
                device_id_type=pl.DeviceIdType.MESH,
            )

        q0r = lax.rem(q + G - 1, G)
        q0l = lax.rem(q + 1, G)
        for s in range(SA):
            rows = pl.ds(s * SUB, SUB)
            a_r[G - 1, rows, :] = t_ref[
                pl.ds(q0r * quarter + s * SUB, SUB), :
            ].astype(jnp.bfloat16)
            a_rdma(a_r, a_send_r, a_recv_r, p_right, 0, s).start()
            a_l[G - 1, rows, :] = t_ref[
                pl.ds(q0l * quarter + qhalf + s * SUB, SUB), :
            ].astype(jnp.bfloat16)
            a_rdma(a_l, a_send_l, a_recv_l, p_left, 0, s).start()

        w_bf[:, :] = w_ref[:, :].astype(jnp.bfloat16)

        for h in range(G - 1):
            qr = lax.rem(q + 2 * G - h - 2, G)
            ql = lax.rem(q + h + 2, G)
            for s in range(SA):
                rows = pl.ds(s * SUB, SUB)
                a_rdma(a_r, a_send_r, a_recv_r, p_right, h, s).wait_recv()
                a_r[h, rows, :] = (
                    a_r[h, rows, :].astype(jnp.float32)
                    + t_ref[pl.ds(qr * quarter + s * SUB, SUB), :]
                ).astype(jnp.bfloat16)
                if h < G - 2:
                    a_rdma(a_r, a_send_r, a_recv_r, p_right, h + 1, s).start()
                a_rdma(a_l, a_send_l, a_recv_l, p_left, h, s).wait_recv()
                a_l[h, rows, :] = (
                    a_l[h, rows, :].astype(jnp.float32)
                    + t_ref[pl.ds(ql * quarter + qhalf + s * SUB, SUB), :]
                ).astype(jnp.bfloat16)
                if h < G - 2:
                    a_rdma(a_l, a_send_l, a_recv_l, p_left, h + 1, s).start()

        q_buf[0:qhalf, :] = a_r[G - 2, :, :]
        q_buf[qhalf:quarter, :] = a_l[G - 2, :, :]

        def b_rdma(h, s):
            src_slot = G - 1 if h == 0 else h - 1
            return pltpu.make_async_remote_copy(
                src_ref=b_comm.at[src_slot, pl.ds(s * SUB, SUB), :],
                dst_ref=b_comm.at[h, pl.ds(s * SUB, SUB), :],
                send_sem=b_send.at[h, s],
                recv_sem=b_recv.at[h, s],
                device_id=(z_up,),
                device_id_type=pl.DeviceIdType.MESH,
            )

        j0 = lax.rem(zi + G - 1, G)
        for s in range(SB):
            b_comm[G - 1, pl.ds(s * SUB, SUB), :] = q_buf[
                pl.ds(j0 * chunk + s * SUB, SUB), :
            ]
            b_rdma(0, s).start()
        for h in range(G - 1):
            jr = lax.rem(zi + 2 * G - h - 2, G)
            for s in range(SB):
                rows = pl.ds(s * SUB, SUB)
                b_rdma(h, s).wait_recv()
                b_comm[h, rows, :] = (
                    b_comm[h, rows, :].astype(jnp.float32)
                    + q_buf[pl.ds(jr * chunk + s * SUB, SUB), :].astype(jnp.float32)
                ).astype(jnp.bfloat16)
                if h < G - 2:
                    b_rdma(h + 1, s).start()


        def d_rdma(c, g, s):
            rows = pl.ds(c * chunk + s * SUB, SUB)
            return pltpu.make_async_remote_copy(
                src_ref=qo_buf.at[rows, :],
                dst_ref=qo_buf.at[rows, :],
                send_sem=d_send.at[g, s],
                recv_sem=d_recv.at[g, s],
                device_id=(z_up,),
                device_id_type=pl.DeviceIdType.MESH,
            )

        def e_rdma(send_sems, recv_sems, dst, c, off, g, s):
            rows = pl.ds(c * quarter + off + s * SUB, SUB)
            return pltpu.make_async_remote_copy(
                src_ref=po_buf.at[rows, :],
                dst_ref=po_buf.at[rows, :],
                send_sem=send_sems.at[g, s],
                recv_sem=recv_sems.at[g, s],
                device_id=(dst,),
                device_id_type=pl.DeviceIdType.MESH,
            )

        def e0_rdma(send_sems, recv_sems, dst, off, s):
            return pltpu.make_async_remote_copy(
                src_ref=qo_buf.at[pl.ds(off + s * SUB, SUB), :],
                dst_ref=po_buf.at[pl.ds(q * quarter + off + s * SUB, SUB), :],
                send_sem=send_sems.at[0, s],
                recv_sem=recv_sems.at[0, s],
                device_id=(dst,),
                device_id_type=pl.DeviceIdType.MESH,
            )

        def start_e0_for_chunk(x, half_idx=None):
            for s in range(SE):
                if half_idx is not None and ((s * SUB) % chunk) // 64 != half_idx:
                    continue

                @pl.when(x == (s * SUB) // chunk)
                def _():
                    e0_rdma(e_send_r, e_recv_r, p_right, 0, s).start()

                @pl.when(x == 2 + (s * SUB) // chunk)
                def _():
                    e0_rdma(e_send_l, e_recv_l, p_left, qhalf, s).start()

        for half_idx in (0, 1):
            res = jnp.dot(
                b_comm[G - 2, pl.ds(half_idx * 64, 64), :], w_bf[:, :],
                preferred_element_type=jnp.float32,
            )
            out_ref[pl.ds(q * quarter + zi * chunk + half_idx * 64, 64), :] = res
            qo_buf[pl.ds(zi * chunk + half_idx * 64, 64), :] = res.astype(
                jnp.bfloat16
            )
            for s in range(SB):
                if (s * SUB) // 64 == half_idx:
                    d_rdma(zi, 0, s).start()
            start_e0_for_chunk(zi, half_idx)
        for g in range(G - 1):
            jc = lax.rem(zi + 2 * G - g - 1, G)
            for s in range(SB):
                d_rdma(jc, g, s).wait_recv()
                if g < G - 2:
                    d_rdma(jc, g + 1, s).start()
            start_e0_for_chunk(jc)
            for s in range(SB):
                rows_o = pl.ds(q * quarter + jc * chunk + s * SUB, SUB)
                rows_q = pl.ds(jc * chunk + s * SUB, SUB)
                out_ref[rows_o, :] = qo_buf[rows_q, :].astype(jnp.float32)

        for g in range(G - 1):
            cr = lax.rem(q + 2 * G - g - 1, G)
            cl = lax.rem(q + g + 1, G)
            for s in range(SE):
                e_rdma(e_send_r, e_recv_r, p_right, cr, 0, g, s).wait_recv()
                if g < G - 2:
                    e_rdma(e_send_r, e_recv_r, p_right, cr, 0, g + 1, s).start()
                rows_r = pl.ds(cr * quarter + s * SUB, SUB)
                out_ref[rows_r, :] = po_buf[rows_r, :].astype(jnp.float32)
                e_rdma(e_send_l, e_recv_l, p_left, cl, qhalf, g, s).wait_recv()
                if g < G - 2:
                    e_rdma(e_send_l, e_recv_l, p_left, cl, qhalf, g + 1, s).start()
                rows_l = pl.ds(cl * quarter + qhalf + s * SUB, SUB)
                out_ref[rows_l, :] = po_buf[rows_l, :].astype(jnp.float32)

        for h in range(G - 1):
            for s in range(SA):
                a_rdma(a_r, a_send_r, a_recv_r, p_right, h, s).wait_send()
                a_rdma(a_l, a_send_l, a_recv_l, p_left, h, s).wait_send()
            for s in range(SB):
                b_rdma(h, s).wait_send()
                d_rdma(zi, h, s).wait_send()
            for s in range(SE):
                e_rdma(e_send_r, e_recv_r, p_right, q, 0, h, s).wait_send()
                e_rdma(e_send_l, e_recv_l, p_left, q, qhalf, h, s).wait_send()

    return pl.pallas_call(
        body,
        out_shape=jax.ShapeDtypeStruct((m, n), jnp.float32),
        in_specs=[
            pl.BlockSpec(memory_space=pltpu.VMEM),
            pl.BlockSpec(memory_space=pltpu.VMEM),
        ],
        out_specs=pl.BlockSpec(memory_space=pltpu.VMEM),
        scratch_shapes=[
            pltpu.VMEM((G, qhalf, k), jnp.bfloat16),
            pltpu.VMEM((G, qhalf, k), jnp.bfloat16),
            pltpu.VMEM((quarter, k), jnp.bfloat16),
            pltpu.VMEM((G, chunk, k), jnp.bfloat16),
            pltpu.VMEM((quarter, n), jnp.bfloat16),
            pltpu.VMEM((m, n), jnp.bfloat16),
            pltpu.VMEM((k, n), jnp.bfloat16),
            pltpu.SemaphoreType.DMA((G - 1, SA)),
            pltpu.SemaphoreType.DMA((G - 1, SA)),
            pltpu.SemaphoreType.DMA((G - 1, SA)),
            pltpu.SemaphoreType.DMA((G - 1, SA)),
            pltpu.SemaphoreType.DMA((G - 1, SB)),
            pltpu.SemaphoreType.DMA((G - 1, SB)),
            pltpu.SemaphoreType.DMA((G - 1, SB)),
            pltpu.SemaphoreType.DMA((G - 1, SB)),
            pltpu.SemaphoreType.DMA((G - 1, SE)),
            pltpu.SemaphoreType.DMA((G - 1, SE)),
            pltpu.SemaphoreType.DMA((G - 1, SE)),
            pltpu.SemaphoreType.DMA((G - 1, SE)),
        ],
        compiler_params=pltpu.CompilerParams(collective_id=0),
    )(t, W)
